# baseline (speedup 1.0000x reference)
import functools

import jax
import jax.numpy as jnp
from jax import lax
from jax.experimental import pallas as pl
from jax.experimental.pallas import tpu as pltpu
from jax.experimental.pallas import tpu_sc as plsc

_CHUNK = 128


def _make_gather_kernel(B, D, n_cores, n_subcores):
    nw = n_cores * n_subcores
    b_per_w = B // nw
    n_chunks = b_per_w // _CHUNK

    mesh = plsc.VectorSubcoreMesh(core_axis_name="c", subcore_axis_name="s")

    @functools.partial(
        pl.kernel,
        mesh=mesh,
        compiler_params=pltpu.CompilerParams(use_tc_tiling_on_sc=False),
        out_type=(
            jax.ShapeDtypeStruct((B, D), jnp.float32),
            jax.ShapeDtypeStruct((B, D), jnp.float32),
        ),
        scratch_types=[
            pltpu.VMEM((n_chunks, _CHUNK), jnp.int32),
            pltpu.VMEM((b_per_w, D), jnp.float32),
            pltpu.VMEM((b_per_w, D), jnp.float32),
            pltpu.SemaphoreType.DMA,
        ],
    )
    def gather_kernel(idx_hbm, mu_hbm, ls_hbm, mu_out, ls_out,
                      idx_v, mu_rows, ls_rows, sem):
        wid = lax.axis_index("s") * n_cores + lax.axis_index("c")
        base = pl.multiple_of(wid * b_per_w, b_per_w)
        pltpu.sync_copy(idx_hbm.at[pl.ds(wid * n_chunks, n_chunks)], idx_v)

        copies = []
        for j in range(n_chunks):
            rows = pl.ds(j * _CHUNK, _CHUNK)
            copies.append(pltpu.async_copy(
                mu_hbm.at[idx_v.at[j]], mu_rows.at[rows], sem))
            copies.append(pltpu.async_copy(
                ls_hbm.at[idx_v.at[j]], ls_rows.at[rows], sem))
        for cp in copies:
            cp.wait()

        out_rows = pl.ds(base, b_per_w)
        pltpu.sync_copy(mu_rows, mu_out.at[out_rows])
        pltpu.sync_copy(ls_rows, ls_out.at[out_rows])

    return gather_kernel


def kernel(indices, mu, log_sigma):
    B = indices.shape[0]
    N, D = mu.shape
    info = plsc.get_sparse_core_info()
    gather = _make_gather_kernel(B, D, info.num_cores, info.num_subcores)
    idx2d = indices.astype(jnp.int32).reshape(B // _CHUNK, _CHUNK)
    return gather(idx2d, mu, log_sigma)


# speedup vs baseline: 2.3945x; 2.3945x over previous
import functools

import jax
import jax.numpy as jnp
from jax import lax
from jax.experimental import pallas as pl
from jax.experimental.pallas import tpu as pltpu
from jax.experimental.pallas import tpu_sc as plsc

_SUB = 8


def _make_gather_kernel(B, D, n_cores, n_subcores):
    nw = n_cores * n_subcores
    b_per_w = B // nw
    half = b_per_w // 2
    ht = half // _SUB

    mesh = plsc.VectorSubcoreMesh(core_axis_name="c", subcore_axis_name="s")

    @functools.partial(
        pl.kernel,
        mesh=mesh,
        out_type=(
            jax.ShapeDtypeStruct((B // _SUB, _SUB, D), jnp.float32),
            jax.ShapeDtypeStruct((B // _SUB, _SUB, D), jnp.float32),
        ),
        scratch_types=[
            pltpu.VMEM((b_per_w,), jnp.int32),
            pltpu.VMEM((ht, _SUB, D), jnp.float32),
            pltpu.VMEM((ht, _SUB, D), jnp.float32),
            pltpu.SemaphoreType.DMA,
        ],
    )
    def gather_kernel(idx_hbm, mu_hbm, ls_hbm, mu_out, ls_out,
                      idx_v, mu_st, ls_st, sem):
        wid = lax.axis_index("s") * n_cores + lax.axis_index("c")
        base = pl.multiple_of(wid * b_per_w, b_per_w)
        pltpu.sync_copy(idx_hbm.at[pl.ds(base, b_per_w)], idx_v)

        grp = 16
        n_grp = half // grp
        qbase = wid * (b_per_w // _SUB)

        for h in range(2):
            def issue(g, carry):
                v = idx_v[pl.ds(h * half + g * grp, grp)]
                tv = lax.div(v, _SUB)
                sv = lax.rem(v, _SUB)
                for j in range(grp):
                    t = tv[j]
                    s = sv[j]
                    q = g * (grp // _SUB) + j // _SUB
                    m = j % _SUB
                    pltpu.async_copy(mu_hbm.at[t, s], mu_st.at[q, m], sem)
                    pltpu.async_copy(ls_hbm.at[t, s], ls_st.at[q, m], sem)
                return carry

            lax.fori_loop(0, n_grp, issue, 0)
            pltpu.make_async_copy(mu_hbm.at[pl.ds(0, ht)], mu_st, sem).wait()
            pltpu.make_async_copy(ls_hbm.at[pl.ds(0, ht)], ls_st, sem).wait()

            out_sl = pl.ds(qbase + h * ht, ht)
            pltpu.sync_copy(mu_st, mu_out.at[out_sl])
            pltpu.sync_copy(ls_st, ls_out.at[out_sl])

    return gather_kernel


def kernel(indices, mu, log_sigma):
    B = indices.shape[0]
    N, D = mu.shape
    info = plsc.get_sparse_core_info()
    gather = _make_gather_kernel(B, D, info.num_cores, info.num_subcores)
    mu3 = mu.reshape(N // _SUB, _SUB, D)
    ls3 = log_sigma.reshape(N // _SUB, _SUB, D)
    mu_out, ls_out = gather(indices.astype(jnp.int32), mu3, ls3)
    return (mu_out.reshape(B, D), ls_out.reshape(B, D))
